# baseline (speedup 1.0000x reference)
import functools

import jax
import jax.numpy as jnp
from jax import lax
from jax.experimental import pallas as pl
from jax.experimental.pallas import tpu as pltpu
from jax.experimental.pallas import tpu_sc as plsc

EMB_DIM = 64
IDX_W = 128
NSLOT = 4


@functools.partial(jax.jit, static_argnames=("idx_per_worker",))
def _embed_lookup(x_idx, table, idx_per_worker):
    mesh = plsc.VectorSubcoreMesh(core_axis_name="c", subcore_axis_name="s")
    info = plsc.get_sparse_core_info()
    nc = info.num_cores

    n_chunks = idx_per_worker // IDX_W
    n_outer = n_chunks // NSLOT

    def body(idx_hbm, table_hbm, out_hbm, idx_v, rows_v, rowsT_v, sem_g,
             sem_s):
        wid = lax.axis_index("s") * nc + lax.axis_index("c")
        base = wid * idx_per_worker
        pltpu.sync_copy(idx_hbm.at[pl.ds(base, idx_per_worker)], idx_v)

        def fire_gather(c, slot):
            pltpu.async_copy(table_hbm.at[idx_v.at[pl.ds(c * IDX_W, IDX_W)]],
                             rows_v.at[slot], sem_g.at[slot])

        def wait_gather(slot):
            pltpu.make_async_copy(table_hbm.at[idx_v.at[pl.ds(0, IDX_W)]],
                                  rows_v.at[slot], sem_g.at[slot]).wait()

        def fire_store(c, slot):
            g = base // IDX_W + c
            pltpu.async_copy(rowsT_v.at[slot],
                             out_hbm.at[g // 32, :, g % 32],
                             sem_s.at[slot])

        def wait_store(slot):
            pltpu.make_async_copy(rowsT_v.at[slot],
                                  out_hbm.at[0, :, 0],
                                  sem_s.at[slot]).wait()

        lane = lax.iota(jnp.int32, 16)

        def transpose(slot):
            src = rows_v.at[slot]

            def trans_d(d, carry):
                dvec = jnp.full((16,), d, jnp.int32)
                dst_row = rowsT_v.at[slot, d // 8, d % 8]
                for k in range(8):
                    b0 = k * 16
                    dst_row[pl.ds(b0, 16)] = plsc.load_gather(
                        src, [lane + b0, dvec])
                return carry

            lax.fori_loop(0, EMB_DIM, trans_d, 0)

        for c in range(NSLOT):
            fire_gather(c, c)

        def outer(o, carry):
            for b in range(NSLOT):
                i = o * NSLOT + b
                wait_gather(b)

                @pl.when(o > 0)
                def _():
                    wait_store(b)

                transpose(b)
                fire_store(i, b)

                @pl.when(o < n_outer - 1)
                def _():
                    fire_gather(i + NSLOT, b)
            return carry

        lax.fori_loop(0, n_outer, outer, 0)

        for b in range(NSLOT):
            wait_store(b)

    run = pl.kernel(
        body,
        out_type=jax.ShapeDtypeStruct((200, 8, 32, 8, 128), jnp.float32),
        mesh=mesh,
        scratch_types=[
            pltpu.VMEM((idx_per_worker,), jnp.int32),
            pltpu.VMEM((NSLOT, IDX_W, EMB_DIM), jnp.float32),
            pltpu.VMEM((NSLOT, 8, 8, IDX_W), jnp.float32),
            pltpu.SemaphoreType.DMA((NSLOT,)),
            pltpu.SemaphoreType.DMA((NSLOT,)),
        ],
        compiler_params=pltpu.CompilerParams(use_tc_tiling_on_sc=False,
                                             needs_layout_passes=False),
    )
    return run(x_idx, table)


def kernel(x, table):
    b, l = x.shape
    total = b * l
    x_idx = x.T.reshape(total).astype(jnp.int32)
    info = plsc.get_sparse_core_info()
    n_workers = info.num_cores * info.num_subcores
    idx_per_worker = total // n_workers
    out5 = _embed_lookup(x_idx, table, idx_per_worker)
    out = (out5.transpose(0, 1, 3, 2, 4)
           .reshape(l, EMB_DIM, b)
           .transpose(2, 0, 1))
    return out


# speedup vs baseline: 2.4199x; 2.4199x over previous
import functools

import jax
import jax.numpy as jnp
from jax import lax
from jax.experimental import pallas as pl
from jax.experimental.pallas import tpu as pltpu
from jax.experimental.pallas import tpu_sc as plsc

EMB_DIM = 64
IDX_W = 128
NSLOT = 4


@functools.partial(jax.jit, static_argnames=("idx_per_worker",))
def _embed_lookup(x_idx, table, idx_per_worker):
    mesh = plsc.VectorSubcoreMesh(core_axis_name="c", subcore_axis_name="s")
    info = plsc.get_sparse_core_info()
    nc = info.num_cores

    n_chunks = idx_per_worker // IDX_W
    n_outer = n_chunks // NSLOT

    def body(idx_hbm, table_hbm, out_hbm, idx_v, rows_v, rowsT_v, sem_g,
             sem_s):
        wid = lax.axis_index("s") * nc + lax.axis_index("c")
        base = wid * idx_per_worker
        pltpu.sync_copy(idx_hbm.at[pl.ds(base, idx_per_worker)], idx_v)

        def fire_gather(c, slot):
            pltpu.async_copy(table_hbm.at[idx_v.at[pl.ds(c * IDX_W, IDX_W)]],
                             rows_v.at[slot], sem_g.at[slot])

        def wait_gather(slot):
            pltpu.make_async_copy(table_hbm.at[idx_v.at[pl.ds(0, IDX_W)]],
                                  rows_v.at[slot], sem_g.at[slot]).wait()

        def fire_store(c, slot):
            g = base // IDX_W + c
            pltpu.async_copy(rowsT_v.at[slot, :, :, pl.ds(0, IDX_W)],
                             out_hbm.at[g // 32, :, g % 32],
                             sem_s.at[slot])

        def wait_store(slot):
            pltpu.make_async_copy(rowsT_v.at[slot, :, :, pl.ds(0, IDX_W)],
                                  out_hbm.at[0, :, 0],
                                  sem_s.at[slot]).wait()

        lane = lax.iota(jnp.int32, 16)
        d_hi = [(lane + k * 16) // 8 for k in range(4)]
        d_lo = lane % 8

        def transpose(slot):
            src = rows_v.at[slot]
            dstT = rowsT_v.at[slot]

            @plsc.parallel_loop(0, IDX_W, step=1, unroll=4)
            def _(b):
                bvec = jnp.full((16,), b, jnp.int32)
                row = src.at[b]
                for k in range(4):
                    plsc.store_scatter(dstT, [d_hi[k], d_lo, bvec],
                                       row[pl.ds(k * 16, 16)])

        for c in range(NSLOT):
            fire_gather(c, c)

        def outer(o, carry):
            for b in range(NSLOT):
                i = o * NSLOT + b
                wait_gather(b)

                @pl.when(o > 0)
                def _():
                    wait_store(b)

                transpose(b)
                fire_store(i, b)

                @pl.when(o < n_outer - 1)
                def _():
                    fire_gather(i + NSLOT, b)
            return carry

        lax.fori_loop(0, n_outer, outer, 0)

        for b in range(NSLOT):
            wait_store(b)

    run = pl.kernel(
        body,
        out_type=jax.ShapeDtypeStruct((200, 8, 32, 8, 128), jnp.float32),
        mesh=mesh,
        scratch_types=[
            pltpu.VMEM((idx_per_worker,), jnp.int32),
            pltpu.VMEM((NSLOT, IDX_W, EMB_DIM), jnp.float32),
            pltpu.VMEM((NSLOT, 8, 8, IDX_W + 1), jnp.float32),
            pltpu.SemaphoreType.DMA((NSLOT,)),
            pltpu.SemaphoreType.DMA((NSLOT,)),
        ],
        compiler_params=pltpu.CompilerParams(use_tc_tiling_on_sc=False,
                                             needs_layout_passes=False),
    )
    return run(x_idx, table)


def kernel(x, table):
    b, l = x.shape
    total = b * l
    x_idx = x.T.reshape(total).astype(jnp.int32)
    info = plsc.get_sparse_core_info()
    n_workers = info.num_cores * info.num_subcores
    idx_per_worker = total // n_workers
    out5 = _embed_lookup(x_idx, table, idx_per_worker)
    out = (out5.transpose(0, 1, 3, 2, 4)
           .reshape(l, EMB_DIM, b)
           .transpose(2, 0, 1))
    return out
